# baseline (speedup 1.0000x reference)
import jax
import jax.numpy as jnp
from jax import lax
from jax.experimental import pallas as pl
from jax.experimental.pallas import tpu as pltpu
from jax.experimental.pallas import tpu_sc as plsc


_NC = 2
_CHUNK_ROWS = 512


def _scs_copy_body(w_hbm, o_hbm, buf0, buf1, lsem0, lsem1, ssem0, ssem1):
    cid = lax.axis_index("c")
    rows_per_c = w_hbm.shape[0] // _NC
    n = rows_per_c // _CHUNK_ROWS
    base = cid * rows_per_c
    bufs = (buf0, buf1)
    lsems = (lsem0, lsem1)
    ssems = (ssem0, ssem1)
    loads = [
        pltpu.make_async_copy(
            w_hbm.at[pl.ds(base + i * _CHUNK_ROWS, _CHUNK_ROWS), :],
            bufs[i % 2],
            lsems[i % 2],
        )
        for i in range(n)
    ]
    stores = [
        pltpu.make_async_copy(
            bufs[i % 2],
            o_hbm.at[0, pl.ds(base + i * _CHUNK_ROWS, _CHUNK_ROWS), :],
            ssems[i % 2],
        )
        for i in range(n)
    ]
    loads[0].start()
    for i in range(n):
        if i + 1 < n:
            if i >= 1:
                stores[i - 1].wait()
            loads[i + 1].start()
        loads[i].wait()
        stores[i].start()
    stores[n - 1].wait()
    if n >= 2:
        stores[n - 2].wait()


def kernel(x, emb_weight):
    seq_len = x.shape[1]
    dim = emb_weight.shape[1]
    sc_copy = pl.kernel(
        _scs_copy_body,
        out_type=jax.ShapeDtypeStruct((1, seq_len, dim), emb_weight.dtype),
        mesh=plsc.ScalarSubcoreMesh(axis_name="c", num_cores=_NC),
        scratch_types=[
            pltpu.VMEM_SHARED((_CHUNK_ROWS, dim), emb_weight.dtype),
            pltpu.VMEM_SHARED((_CHUNK_ROWS, dim), emb_weight.dtype),
            pltpu.SemaphoreType.DMA,
            pltpu.SemaphoreType.DMA,
            pltpu.SemaphoreType.DMA,
            pltpu.SemaphoreType.DMA,
        ],
    )
    return sc_copy(emb_weight)


# speedup vs baseline: 2.2379x; 2.2379x over previous
import jax
import jax.numpy as jnp
from jax.experimental import pallas as pl
from jax.experimental.pallas import tpu as pltpu


_BLOCK_ROWS = 2048


def _copy_body(w_ref, o_ref):
    o_ref[...] = w_ref[...][None]


def kernel(x, emb_weight):
    seq_len = x.shape[1]
    dim = emb_weight.shape[1]
    grid = (seq_len // _BLOCK_ROWS,)
    out = pl.pallas_call(
        _copy_body,
        grid=grid,
        out_shape=jax.ShapeDtypeStruct((1, seq_len, dim), emb_weight.dtype),
        in_specs=[pl.BlockSpec((_BLOCK_ROWS, dim), lambda i: (i, 0))],
        out_specs=pl.BlockSpec((1, _BLOCK_ROWS, dim), lambda i: (0, i, 0)),
        compiler_params=pltpu.CompilerParams(
            dimension_semantics=("parallel",)
        ),
    )(emb_weight)
    return out
